# Initial kernel scaffold; baseline (speedup 1.0000x reference)
import jax
import jax.numpy as jnp
from jax.experimental import pallas as pl


def kernel(hidden, gamma, beta, W1, b1, W2, b2):
    raise NotImplementedError("write your pallas kernel here")


# speedup vs baseline: 1.6649x; 1.6649x over previous
import functools

import jax
import jax.numpy as jnp
from jax.experimental import pallas as pl

N = 32768
IN_DIM = 768
HID = 512
E = 64
TOPK = 8
EPS_LN = 1e-5

BLOCK = 512


def _gate_kernel(x_ref, gamma_ref, beta_ref, w1_ref, b1_ref, w2_ref, b2_ref,
                 idx_ref, w_ref):
    x = x_ref[...]
    mu = jnp.mean(x, axis=-1, keepdims=True)
    var = jnp.mean(jnp.square(x), axis=-1, keepdims=True) - jnp.square(mu)
    xn = (x - mu) * jax.lax.rsqrt(var + EPS_LN)
    xn = xn * gamma_ref[...] + beta_ref[...]
    h = jnp.dot(xn, w1_ref[...], preferred_element_type=jnp.float32)
    h = h + b1_ref[...]
    h = 0.5 * h * (1.0 + jax.lax.erf(h * 0.7071067811865476))
    logits = jnp.dot(h, w2_ref[...], preferred_element_type=jnp.float32)
    logits = logits + b2_ref[...]
    m = jnp.max(logits, axis=-1, keepdims=True)
    e = jnp.exp(logits - m)
    lane = jax.lax.broadcasted_iota(jnp.int32, e.shape, 1)
    work = e
    idx_cols = []
    val_cols = []
    for _ in range(TOPK):
        v = jnp.max(work, axis=-1, keepdims=True)
        cand = jnp.where(work == v, lane, E)
        i = jnp.min(cand, axis=-1, keepdims=True)
        idx_cols.append(i)
        val_cols.append(v)
        work = jnp.where(lane == i, -1.0, work)
    vals = jnp.concatenate(val_cols, axis=-1)
    idxs = jnp.concatenate(idx_cols, axis=-1)
    denom = jnp.sum(vals, axis=-1, keepdims=True) + 1e-20
    idx_ref[...] = idxs
    w_ref[...] = vals / denom


@jax.jit
def kernel(hidden, gamma, beta, W1, b1, W2, b2):
    gamma2 = gamma.reshape(1, IN_DIM)
    beta2 = beta.reshape(1, IN_DIM)
    b1_2 = b1.reshape(1, HID)
    b2_2 = b2.reshape(1, E)
    grid = (N // BLOCK,)
    full = lambda shape: pl.BlockSpec(shape, lambda i: (0, 0))
    out = pl.pallas_call(
        _gate_kernel,
        grid=grid,
        in_specs=[
            pl.BlockSpec((BLOCK, IN_DIM), lambda i: (i, 0)),
            full((1, IN_DIM)),
            full((1, IN_DIM)),
            full((IN_DIM, HID)),
            full((1, HID)),
            full((HID, E)),
            full((1, E)),
        ],
        out_specs=[
            pl.BlockSpec((BLOCK, TOPK), lambda i: (i, 0)),
            pl.BlockSpec((BLOCK, TOPK), lambda i: (i, 0)),
        ],
        out_shape=[
            jax.ShapeDtypeStruct((N, TOPK), jnp.int32),
            jax.ShapeDtypeStruct((N, TOPK), jnp.float32),
        ],
    )(hidden, gamma2, beta2, W1, b1_2, W2, b2_2)
    return out[0], out[1]


# speedup vs baseline: 2.9485x; 1.7710x over previous
import functools

import jax
import jax.numpy as jnp
from jax.experimental import pallas as pl

N = 32768
IN_DIM = 768
HID = 512
E = 64
TOPK = 8
EPS_LN = 1e-5

BLOCK = 512


def _gate_kernel(x_ref, gamma_ref, beta_ref, w1_ref, b1_ref, w2_ref, b2_ref,
                 idx_ref, w_ref):
    x = x_ref[...]
    mu = jnp.mean(x, axis=-1, keepdims=True)
    var = jnp.mean(jnp.square(x), axis=-1, keepdims=True) - jnp.square(mu)
    xn = (x - mu) * jax.lax.rsqrt(var + EPS_LN)
    xn = xn * gamma_ref[...] + beta_ref[...]
    h = jnp.dot(xn, w1_ref[...], preferred_element_type=jnp.float32)
    h = h + b1_ref[...]
    h = 0.5 * h * (1.0 + jax.lax.erf(h * 0.7071067811865476))
    logits_t = jax.lax.dot_general(
        w2_ref[...], h, (((0,), (1,)), ((), ())),
        preferred_element_type=jnp.float32)
    logits_t = logits_t + b2_ref[...]
    m = jnp.max(logits_t, axis=0, keepdims=True)
    e = jnp.exp(logits_t - m)
    sub = jax.lax.broadcasted_iota(jnp.int32, e.shape, 0)
    work = e
    idx_rows = []
    val_rows = []
    for _ in range(TOPK):
        v = jnp.max(work, axis=0, keepdims=True)
        cand = jnp.where(work == v, sub, E)
        i = jnp.min(cand, axis=0, keepdims=True)
        idx_rows.append(i)
        val_rows.append(v)
        work = jnp.where(sub == i, -1.0, work)
    vals = jnp.concatenate(val_rows, axis=0)
    idxs = jnp.concatenate(idx_rows, axis=0)
    denom = jnp.sum(vals, axis=0, keepdims=True) + 1e-20
    w_t = vals / denom
    idx_ref[...] = idxs.T
    w_ref[...] = w_t.T


@jax.jit
def kernel(hidden, gamma, beta, W1, b1, W2, b2):
    gamma2 = gamma.reshape(1, IN_DIM)
    beta2 = beta.reshape(1, IN_DIM)
    b1_2 = b1.reshape(1, HID)
    b2_2 = b2.reshape(E, 1)
    grid = (N // BLOCK,)
    full = lambda shape: pl.BlockSpec(shape, lambda i: (0, 0))
    out = pl.pallas_call(
        _gate_kernel,
        grid=grid,
        in_specs=[
            pl.BlockSpec((BLOCK, IN_DIM), lambda i: (i, 0)),
            full((1, IN_DIM)),
            full((1, IN_DIM)),
            full((IN_DIM, HID)),
            full((1, HID)),
            full((HID, E)),
            full((E, 1)),
        ],
        out_specs=[
            pl.BlockSpec((BLOCK, TOPK), lambda i: (i, 0)),
            pl.BlockSpec((BLOCK, TOPK), lambda i: (i, 0)),
        ],
        out_shape=[
            jax.ShapeDtypeStruct((N, TOPK), jnp.int32),
            jax.ShapeDtypeStruct((N, TOPK), jnp.float32),
        ],
    )(hidden, gamma2, beta2, W1, b1_2, W2, b2_2)
    return out[0], out[1]


# speedup vs baseline: 3.0017x; 1.0180x over previous
import functools

import jax
import jax.numpy as jnp
from jax.experimental import pallas as pl

N = 32768
IN_DIM = 768
HID = 512
E = 64
TOPK = 8
EPS_LN = 1e-5

BLOCK = 512


def _gate_kernel(x_ref, gamma_ref, beta_ref, w1_ref, b1_ref, w2_ref, b2_ref,
                 idx_ref, w_ref):
    x = x_ref[...]
    mu = jnp.mean(x, axis=-1, keepdims=True)
    var = jnp.mean(jnp.square(x), axis=-1, keepdims=True) - jnp.square(mu)
    xn = (x - mu) * jax.lax.rsqrt(var + EPS_LN)
    xn = xn * gamma_ref[...] + beta_ref[...]
    h = jnp.dot(xn, w1_ref[...], preferred_element_type=jnp.float32)
    h = h + b1_ref[...]
    h = 0.5 * h * (1.0 + jax.lax.erf(h * 0.7071067811865476))
    logits_t = jax.lax.dot_general(
        w2_ref[...], h, (((0,), (1,)), ((), ())),
        preferred_element_type=jnp.float32)
    logits_t = logits_t + b2_ref[...]
    sub = jax.lax.broadcasted_iota(jnp.int32, logits_t.shape, 0)
    work = logits_t
    idx_rows = []
    val_rows = []
    for _ in range(TOPK):
        v = jnp.max(work, axis=0, keepdims=True)
        cand = jnp.where(work == v, sub, E)
        i = jnp.min(cand, axis=0, keepdims=True)
        idx_rows.append(i)
        val_rows.append(v)
        work = jnp.where(sub == i, -jnp.inf, work)
    lsel = jnp.concatenate(val_rows, axis=0)
    idxs = jnp.concatenate(idx_rows, axis=0)
    vals = jnp.exp(lsel - lsel[0:1, :])
    denom = jnp.sum(vals, axis=0, keepdims=True) + 1e-20
    w_t = vals / denom
    idx_ref[...] = idxs.T
    w_ref[...] = w_t.T


@jax.jit
def kernel(hidden, gamma, beta, W1, b1, W2, b2):
    gamma2 = gamma.reshape(1, IN_DIM)
    beta2 = beta.reshape(1, IN_DIM)
    b1_2 = b1.reshape(1, HID)
    b2_2 = b2.reshape(E, 1)
    grid = (N // BLOCK,)
    full = lambda shape: pl.BlockSpec(shape, lambda i: (0, 0))
    out = pl.pallas_call(
        _gate_kernel,
        grid=grid,
        in_specs=[
            pl.BlockSpec((BLOCK, IN_DIM), lambda i: (i, 0)),
            full((1, IN_DIM)),
            full((1, IN_DIM)),
            full((IN_DIM, HID)),
            full((1, HID)),
            full((HID, E)),
            full((E, 1)),
        ],
        out_specs=[
            pl.BlockSpec((BLOCK, TOPK), lambda i: (i, 0)),
            pl.BlockSpec((BLOCK, TOPK), lambda i: (i, 0)),
        ],
        out_shape=[
            jax.ShapeDtypeStruct((N, TOPK), jnp.int32),
            jax.ShapeDtypeStruct((N, TOPK), jnp.float32),
        ],
    )(hidden, gamma2, beta2, W1, b1_2, W2, b2_2)
    return out[0], out[1]
